# baseline (speedup 1.0000x reference)
import functools

import jax
import jax.numpy as jnp
from jax import lax
from jax.experimental import pallas as pl
from jax.experimental.pallas import tpu as pltpu
from jax.experimental.pallas import tpu_sc as plsc


_PIECE = 8192
_NB = 4
_LOOK = 2


class _Ring:
    def __init__(self, buf, sins, souts, src_slice, dst_slice, n):
        self.buf, self.sins, self.souts = buf, sins, souts
        self.src, self.dst, self.n = src_slice, dst_slice, n
        self.gh = [None] * _NB
        self.sh = [None] * _NB

    def _gather(self, k):
        b = k % _NB
        if self.sh[b] is not None:
            self.sh[b].wait()
            self.sh[b] = None
        self.gh[b] = pltpu.async_copy(self.src(k), self.buf.at[b], self.sins[b])

    def prime(self):
        for k in range(min(_LOOK, self.n)):
            self._gather(k)

    def step(self, k):
        b = k % _NB
        self.gh[b].wait()
        self.sh[b] = pltpu.async_copy(self.buf.at[b], self.dst(k), self.souts[b])
        if k + _LOOK < self.n:
            self._gather(k + _LOOK)

    def drain(self):
        for b in range(_NB):
            if self.sh[b] is not None:
                self.sh[b].wait()


def _sc_permute(values, weights, W, C):
    info = plsc.get_sparse_core_info()
    nc, ns = info.num_cores, info.num_subcores
    nw = nc * ns
    pairs = W * W
    per_w = pairs // nw
    npieces = C // _PIECE
    n = per_w * npieces

    mesh = plsc.VectorSubcoreMesh(core_axis_name="c", subcore_axis_name="s")

    @functools.partial(
        pl.kernel,
        mesh=mesh,
        out_type=[
            jax.ShapeDtypeStruct((W, W * C), jnp.int32),
            jax.ShapeDtypeStruct((W, W * C), jnp.float32),
        ],
        scratch_types=[
            pltpu.VMEM((_NB, _PIECE), jnp.int32),
            pltpu.VMEM((_NB, _PIECE), jnp.float32),
        ]
        + [pltpu.SemaphoreType.DMA] * (4 * _NB),
    )
    def k(vals_hbm, wts_hbm, out_i_hbm, out_w_hbm, vbuf, wbuf, *sems):
        v_sin, v_sout = sems[:_NB], sems[_NB : 2 * _NB]
        w_sin, w_sout = sems[2 * _NB : 3 * _NB], sems[3 * _NB :]
        wid = lax.axis_index("s") * nc + lax.axis_index("c")

        coords = []
        for t in range(per_w):
            p = wid * per_w + t
            coords.append((p // W, p % W))

        def src_slice(hbm):
            def f(k):
                s, w = coords[k // npieces]
                j = k % npieces
                return hbm.at[s, w, pl.ds(j * _PIECE, _PIECE)]
            return f

        def dst_slice(hbm):
            def f(k):
                s, w = coords[k // npieces]
                j = k % npieces
                return hbm.at[w, pl.ds(s * C + j * _PIECE, _PIECE)]
            return f

        rings = (
            _Ring(vbuf, v_sin, v_sout, src_slice(vals_hbm), dst_slice(out_i_hbm), n),
            _Ring(wbuf, w_sin, w_sout, src_slice(wts_hbm), dst_slice(out_w_hbm), n),
        )
        for r in rings:
            r.prime()
        for kk in range(n):
            for r in rings:
                r.step(kk)
        for r in rings:
            r.drain()

    return k(values, weights)


def _tc_constants(W, N):
    def body(len_ref, off_ref):
        len_ref[...] = jnp.ones(len_ref.shape, jnp.int32)
        off_ref[...] = lax.broadcasted_iota(jnp.int32, off_ref.shape, 2)

    lengths3, offsets3 = pl.pallas_call(
        body,
        grid=(W,),
        out_specs=[
            pl.BlockSpec((1, 1, N), lambda i: (i, 0, 0)),
            pl.BlockSpec((1, 1, N + 1), lambda i: (i, 0, 0)),
        ],
        out_shape=[
            jax.ShapeDtypeStruct((W, 1, N), jnp.int32),
            jax.ShapeDtypeStruct((W, 1, N + 1), jnp.int32),
        ],
    )()
    return lengths3.reshape(W, N), offsets3.reshape(W, N + 1)


def kernel(values, weights):
    W, _, C = values.shape
    N = W * C
    out_indices, out_weights = _sc_permute(values, weights, W, C)
    kjt_lengths, kjt_offsets = _tc_constants(W, N)
    return out_indices, out_weights, kjt_lengths, kjt_offsets


# speedup vs baseline: 1.0395x; 1.0395x over previous
import functools

import jax
import jax.numpy as jnp
from jax import lax
from jax.experimental import pallas as pl
from jax.experimental.pallas import tpu as pltpu
from jax.experimental.pallas import tpu_sc as plsc


_PIECE = 8192
_NB = 4
_LOOK = 2


class _Ring:
    def __init__(self, buf, sins, souts, src_slice, dst_slice, n):
        self.buf, self.sins, self.souts = buf, sins, souts
        self.src, self.dst, self.n = src_slice, dst_slice, n
        self.gh = [None] * _NB
        self.sh = [None] * _NB

    def _gather(self, k):
        b = k % _NB
        if self.sh[b] is not None:
            self.sh[b].wait()
            self.sh[b] = None
        self.gh[b] = pltpu.async_copy(self.src(k), self.buf.at[b], self.sins[b])

    def prime(self):
        for k in range(min(_LOOK, self.n)):
            self._gather(k)

    def step(self, k):
        b = k % _NB
        self.gh[b].wait()
        self.sh[b] = pltpu.async_copy(self.buf.at[b], self.dst(k), self.souts[b])
        if k + _LOOK < self.n:
            self._gather(k + _LOOK)

    def drain(self):
        for b in range(_NB):
            if self.sh[b] is not None:
                self.sh[b].wait()


def _sc_permute(values, weights, W, C):
    info = plsc.get_sparse_core_info()
    nc, ns = info.num_cores, info.num_subcores
    nw = nc * ns
    pairs = W * W
    per_w = pairs // nw
    npieces = C // _PIECE
    n = per_w * npieces

    mesh = plsc.VectorSubcoreMesh(core_axis_name="c", subcore_axis_name="s")

    @functools.partial(
        pl.kernel,
        mesh=mesh,
        out_type=[
            jax.ShapeDtypeStruct((W, W * C), jnp.int32),
            jax.ShapeDtypeStruct((W, W * C), jnp.float32),
        ],
        scratch_types=[
            pltpu.MemorySpace.VMEM_SHARED((ns, _NB, _PIECE), jnp.int32),
            pltpu.MemorySpace.VMEM_SHARED((ns, _NB, _PIECE), jnp.float32),
        ]
        + [pltpu.SemaphoreType.DMA] * (4 * _NB),
    )
    def k(vals_hbm, wts_hbm, out_i_hbm, out_w_hbm, vbuf_all, wbuf_all, *sems):
        v_sin, v_sout = sems[:_NB], sems[_NB : 2 * _NB]
        w_sin, w_sout = sems[2 * _NB : 3 * _NB], sems[3 * _NB :]
        sid = lax.axis_index("s")
        vbuf = vbuf_all.at[sid]
        wbuf = wbuf_all.at[sid]
        wid = lax.axis_index("s") * nc + lax.axis_index("c")

        coords = []
        for t in range(per_w):
            p = wid * per_w + t
            coords.append((p // W, p % W))

        def src_slice(hbm):
            def f(k):
                s, w = coords[k // npieces]
                j = k % npieces
                return hbm.at[s, w, pl.ds(j * _PIECE, _PIECE)]
            return f

        def dst_slice(hbm):
            def f(k):
                s, w = coords[k // npieces]
                j = k % npieces
                return hbm.at[w, pl.ds(s * C + j * _PIECE, _PIECE)]
            return f

        rings = (
            _Ring(vbuf, v_sin, v_sout, src_slice(vals_hbm), dst_slice(out_i_hbm), n),
            _Ring(wbuf, w_sin, w_sout, src_slice(wts_hbm), dst_slice(out_w_hbm), n),
        )
        for r in rings:
            r.prime()
        for kk in range(n):
            for r in rings:
                r.step(kk)
        for r in rings:
            r.drain()

    return k(values, weights)


def _tc_constants(W, N):
    def body(len_ref, off_ref):
        len_ref[...] = jnp.ones(len_ref.shape, jnp.int32)
        off_ref[...] = lax.broadcasted_iota(jnp.int32, off_ref.shape, 2)

    lengths3, offsets3 = pl.pallas_call(
        body,
        grid=(W,),
        out_specs=[
            pl.BlockSpec((1, 1, N), lambda i: (i, 0, 0)),
            pl.BlockSpec((1, 1, N + 1), lambda i: (i, 0, 0)),
        ],
        out_shape=[
            jax.ShapeDtypeStruct((W, 1, N), jnp.int32),
            jax.ShapeDtypeStruct((W, 1, N + 1), jnp.int32),
        ],
    )()
    return lengths3.reshape(W, N), offsets3.reshape(W, N + 1)


def kernel(values, weights):
    W, _, C = values.shape
    N = W * C
    out_indices, out_weights = _sc_permute(values, weights, W, C)
    kjt_lengths, kjt_offsets = _tc_constants(W, N)
    return out_indices, out_weights, kjt_lengths, kjt_offsets
